# baseline (speedup 1.0000x reference)
import jax
import jax.numpy as jnp
from jax import lax
from jax.experimental import pallas as pl
from jax.experimental.pallas import tpu as pltpu
from jax.experimental.pallas import tpu_sc as plsc

NUM_NODES = 100000
NUM_GRAPHS_K = 1024
NUM_WORKERS = 16
LANES = 16
CHUNK = 6256
NVEC_MAIN = CHUNK // LANES
LAST_BASE = CHUNK * (NUM_WORKERS - 1)
LAST_N = NUM_NODES - LAST_BASE
NVEC_LAST = LAST_N // LANES
NUM_REDUCERS = 8
BINS_PER_RED = NUM_GRAPHS_K // NUM_REDUCERS


def _body(x_hbm, ids_hbm, w_hbm, b_hbm, out_hbm,
          idx_v, x_v, acc_v, part_v, out_v, wb_v, sem, shared):
    sid = lax.axis_index("s")

    for j in range(NUM_GRAPHS_K // LANES):
        acc_v[pl.ds(j * LANES, LANES)] = jnp.zeros((LANES,), jnp.float32)

    def process(base, nvec):
        n = nvec * LANES
        cps = [
            pltpu.async_copy(
                ids_hbm.at[pl.ds(base, n)], idx_v.at[pl.ds(0, n)], sem),
            pltpu.async_copy(
                x_hbm.at[pl.ds(base, n)], x_v.at[pl.ds(0, n)], sem),
            pltpu.async_copy(w_hbm, wb_v.at[pl.ds(0, 1)], sem),
            pltpu.async_copy(b_hbm, wb_v.at[pl.ds(8, 1)], sem),
        ]
        for cp in cps:
            cp.wait()

        pos0 = lax.iota(jnp.int32, LANES) * nvec

        @plsc.parallel_loop(0, nvec, 1, unroll=8, carry=pos0)
        def _(i, pos):
            idx = plsc.load_gather(idx_v, [pos])
            xv = plsc.load_gather(x_v, [pos])
            plsc.addupdate_scatter(acc_v, [idx], xv)
            return pos + 1

    @pl.when(sid < NUM_WORKERS - 1)
    def _():
        process(sid * CHUNK, NVEC_MAIN)

    @pl.when(sid == NUM_WORKERS - 1)
    def _():
        process(LAST_BASE, NVEC_LAST)

    pltpu.sync_copy(acc_v, shared.at[pl.ds(sid * NUM_GRAPHS_K, NUM_GRAPHS_K)])
    plsc.subcore_barrier()

    @pl.when(sid < NUM_REDUCERS)
    def _():
        bin_base = sid * BINS_PER_RED
        cps = [
            pltpu.async_copy(
                shared.at[pl.ds(r * NUM_GRAPHS_K + bin_base, BINS_PER_RED)],
                part_v.at[pl.ds(r * BINS_PER_RED, BINS_PER_RED)], sem)
            for r in range(NUM_WORKERS)
        ]
        for cp in cps:
            cp.wait()
        wbvec = wb_v[pl.ds(0, LANES)]
        wv = wbvec[0]
        bv = wbvec[8]

        for j in range(BINS_PER_RED // LANES):
            def red_body(r, s):
                return s + part_v[pl.ds(r * BINS_PER_RED + j * LANES, LANES)]
            s = lax.fori_loop(0, NUM_WORKERS, red_body,
                              jnp.zeros((LANES,), jnp.float32))
            out_v[pl.ds(j * LANES, LANES)] = s * wv + bv

        pltpu.sync_copy(out_v, out_hbm.at[pl.ds(bin_base, BINS_PER_RED)])


@jax.jit
def _run(xf, ids, w1, b1):
    mesh = plsc.VectorSubcoreMesh(core_axis_name="c", subcore_axis_name="s",
                                  num_cores=1)
    f = pl.kernel(
        _body,
        out_type=jax.ShapeDtypeStruct((NUM_GRAPHS_K,), jnp.float32),
        mesh=mesh,
        compiler_params=pltpu.CompilerParams(
            needs_layout_passes=False,
            skip_device_barrier=True,
            disable_bounds_checks=True,
            disable_semaphore_checks=True,
        ),
        scratch_types=[
            pltpu.VMEM((CHUNK,), jnp.int32),
            pltpu.VMEM((CHUNK,), jnp.float32),
            pltpu.VMEM((NUM_GRAPHS_K,), jnp.float32),
            pltpu.VMEM((NUM_WORKERS * BINS_PER_RED,), jnp.float32),
            pltpu.VMEM((BINS_PER_RED,), jnp.float32),
            pltpu.VMEM((LANES,), jnp.float32),
            pltpu.SemaphoreType.DMA,
            pltpu.VMEM_SHARED((NUM_WORKERS * NUM_GRAPHS_K,), jnp.float32),
        ],
    )
    return f(xf, ids, w1, b1)


def kernel(x, batch_ids, W, b):
    xf = x.reshape(NUM_NODES)
    ids = batch_ids.astype(jnp.int32)
    out = _run(xf, ids, W.reshape(1), b.reshape(1))
    return out.reshape(NUM_GRAPHS_K, 1)


# speedup vs baseline: 1.0028x; 1.0028x over previous
import jax
import jax.numpy as jnp
from jax import lax
from jax.experimental import pallas as pl
from jax.experimental.pallas import tpu as pltpu
from jax.experimental.pallas import tpu_sc as plsc

NUM_NODES = 100000
NUM_GRAPHS_K = 1024
NUM_WORKERS = 16
LANES = 16
CHUNK = 6256
NVEC_MAIN = CHUNK // LANES
LAST_BASE = CHUNK * (NUM_WORKERS - 1)
LAST_N = NUM_NODES - LAST_BASE
NVEC_LAST = LAST_N // LANES
NUM_REDUCERS = 8
BINS_PER_RED = NUM_GRAPHS_K // NUM_REDUCERS


def _body(x_hbm, ids_hbm, w_hbm, b_hbm, out_hbm,
          idx_v, x_v, acc_v, part_v, out_v, wb_v, sem, shared):
    sid = lax.axis_index("s")

    for j in range(NUM_GRAPHS_K // LANES):
        acc_v[pl.ds(j * LANES, LANES)] = jnp.zeros((LANES,), jnp.float32)

    def process(base, nvec):
        n = nvec * LANES
        cps = [
            pltpu.async_copy(
                ids_hbm.at[pl.ds(base, n)], idx_v.at[pl.ds(0, n)], sem),
            pltpu.async_copy(
                x_hbm.at[pl.ds(base, n)], x_v.at[pl.ds(0, n)], sem),
            pltpu.async_copy(w_hbm, wb_v.at[pl.ds(0, 1)], sem),
            pltpu.async_copy(b_hbm, wb_v.at[pl.ds(8, 1)], sem),
        ]
        for cp in cps:
            cp.wait()

        pos0 = lax.iota(jnp.int32, LANES) * nvec

        @plsc.parallel_loop(0, nvec, 1, unroll=8, carry=pos0)
        def _(i, pos):
            idx = plsc.load_gather(idx_v, [pos])
            xv = plsc.load_gather(x_v, [pos])
            plsc.addupdate_scatter(acc_v, [idx], xv)
            return pos + 1

    @pl.when(sid < NUM_WORKERS - 1)
    def _():
        process(sid * CHUNK, NVEC_MAIN)

    @pl.when(sid == NUM_WORKERS - 1)
    def _():
        process(LAST_BASE, NVEC_LAST)

    pltpu.sync_copy(acc_v, shared.at[pl.ds(sid * NUM_GRAPHS_K, NUM_GRAPHS_K)])
    plsc.subcore_barrier()

    @pl.when(sid < NUM_REDUCERS)
    def _():
        bin_base = sid * BINS_PER_RED
        cps = [
            pltpu.async_copy(
                shared.at[pl.ds(r * NUM_GRAPHS_K + bin_base, BINS_PER_RED)],
                part_v.at[pl.ds(r * BINS_PER_RED, BINS_PER_RED)], sem)
            for r in range(NUM_WORKERS)
        ]
        for cp in cps:
            cp.wait()
        wbvec = wb_v[pl.ds(0, LANES)]
        wv = wbvec[0]
        bv = wbvec[8]

        for j in range(BINS_PER_RED // LANES):
            def red_body(r, s):
                return s + part_v[pl.ds(r * BINS_PER_RED + j * LANES, LANES)]
            s = lax.fori_loop(0, NUM_WORKERS, red_body,
                              jnp.zeros((LANES,), jnp.float32))
            out_v[pl.ds(j * LANES, LANES)] = s * wv + bv

        pltpu.sync_copy(out_v, out_hbm.at[pl.ds(bin_base, BINS_PER_RED)])


@jax.jit
def _run(xf, ids, w1, b1):
    mesh = plsc.VectorSubcoreMesh(core_axis_name="c", subcore_axis_name="s",
                                  num_cores=1)
    f = pl.kernel(
        _body,
        out_type=jax.ShapeDtypeStruct((NUM_GRAPHS_K,), jnp.float32),
        mesh=mesh,
        compiler_params=pltpu.CompilerParams(needs_layout_passes=False),
        scratch_types=[
            pltpu.VMEM((CHUNK,), jnp.int32),
            pltpu.VMEM((CHUNK,), jnp.float32),
            pltpu.VMEM((NUM_GRAPHS_K,), jnp.float32),
            pltpu.VMEM((NUM_WORKERS * BINS_PER_RED,), jnp.float32),
            pltpu.VMEM((BINS_PER_RED,), jnp.float32),
            pltpu.VMEM((LANES,), jnp.float32),
            pltpu.SemaphoreType.DMA,
            pltpu.VMEM_SHARED((NUM_WORKERS * NUM_GRAPHS_K,), jnp.float32),
        ],
    )
    return f(xf, ids, w1, b1)


def kernel(x, batch_ids, W, b):
    xf = x.reshape(NUM_NODES)
    ids = batch_ids.astype(jnp.int32)
    out = _run(xf, ids, W.reshape(1), b.reshape(1))
    return out.reshape(NUM_GRAPHS_K, 1)
